# baseline (speedup 1.0000x reference)
import functools

import jax
import jax.numpy as jnp
from jax import lax
from jax.experimental import pallas as pl
from jax.experimental.pallas import tpu as pltpu
from jax.experimental.pallas import tpu_sc as plsc

VOCAB = 1000000
VOCAB_PAD = 1000064
EMBED_DIM = 32
BATCH = 4096
SEQ = 200

N = BATCH * SEQ
NC = 2
NS = 16
NW = NC * NS
PER_W = N // NW
CHUNK = 128
NCHUNK = PER_W // CHUNK
K = 8
NSUP = NCHUNK // K

PREP_GRID = 977


def _prep_body(in_ref, out_ref):
    rows = []
    for jj in range(8):
        parts = []
        for w in range(4):
            sub = in_ref[:, jj * 128 + 32 * w : jj * 128 + 32 * w + 32]
            parts.append(sub.T)
        rows.append(jnp.concatenate(parts, axis=1))
    out_ref[...] = jnp.concatenate(rows, axis=0)


def _prep_table(table_t):
    return pl.pallas_call(
        _prep_body,
        grid=(PREP_GRID,),
        in_specs=[pl.BlockSpec((32, 1024), lambda j: (0, j))],
        out_specs=pl.BlockSpec((256, 128), lambda j: (j, 0)),
        out_shape=jax.ShapeDtypeStruct((VOCAB_PAD // 4, 128), jnp.float32),
    )(table_t)


def _post_body(in_ref, out_ref):
    d = in_ref[0, 0].T
    for i_tile in range(4):
        parts = [
            d[32 * w + 8 * i_tile : 32 * w + 8 * i_tile + 8, :]
            for w in range(4)
        ]
        out_ref[0, i_tile, 0] = jnp.concatenate(parts, axis=1)


def _post_tiles(lin):
    return pl.pallas_call(
        _post_body,
        grid=(SEQ, 32),
        in_specs=[pl.BlockSpec((1, 1, 32, 128), lambda s, j: (s, j, 0, 0))],
        out_specs=pl.BlockSpec(
            (1, 4, 1, 8, 128), lambda s, j: (s, 0, j, 0, 0)
        ),
        out_shape=jax.ShapeDtypeStruct((SEQ, 4, 32, 8, 128), jnp.float32),
    )(lin.reshape(SEQ, 32, 32, 128))


def _emb_body(x_hbm, table_hbm, out_hbm, idx_v, rows_v, gsem, wsem):
    cid = lax.axis_index("c")
    sid = lax.axis_index("s")
    wid = sid * NC + cid
    cbase = wid * NCHUNK

    pltpu.sync_copy(x_hbm.at[pl.ds(cbase, NCHUNK)], idx_v)

    def xform_body(t, carry):
        for q in range(8):
            r = idx_v[t, pl.ds(16 * q, 16)]
            m7 = lax.rem(r, 128)
            slot = (r - m7) + lax.shift_left(r & 31, 2) + lax.shift_right_logical(m7, 5)
            idx_v[t, pl.ds(16 * q, 16)] = slot
        return carry

    lax.fori_loop(0, NCHUNK, xform_body, 0)

    def sup_body(s, carry):
        p = lax.rem(s, 2)

        @pl.when(s >= 2)
        def _():
            for k in range(K):
                for a in range(4):
                    pltpu.make_async_copy(
                        rows_v.at[p, k, pl.ds(32 * a, 32)],
                        out_hbm.at[0, :, a],
                        wsem,
                    ).wait()

        for k in range(K):
            pltpu.async_copy(
                table_hbm.at[idx_v.at[s * K + k]], rows_v.at[p, k], gsem
            )
        for k in range(K):
            pltpu.make_async_copy(
                table_hbm.at[idx_v.at[s * K + k]], rows_v.at[p, k], gsem
            ).wait()
        for k in range(K):
            c = cbase + s * K + k
            for a in range(4):
                pltpu.async_copy(
                    rows_v.at[p, k, pl.ds(32 * a, 32)],
                    out_hbm.at[c, :, a],
                    wsem,
                )
        return carry

    lax.fori_loop(0, NSUP, sup_body, 0)

    for _ in range(2 * K * 4):
        pltpu.make_async_copy(
            rows_v.at[0, 0, pl.ds(0, 32)], out_hbm.at[0, :, 0], wsem
        ).wait()


@jax.jit
def kernel(x, table):
    idx = jnp.transpose(x).reshape(N // CHUNK, CHUNK).astype(jnp.int32)
    table_lin = _prep_table(jnp.transpose(table)).reshape(VOCAB_PAD, EMBED_DIM)
    mesh = plsc.VectorSubcoreMesh(core_axis_name="c", subcore_axis_name="s")
    f = functools.partial(
        pl.kernel,
        mesh=mesh,
        out_type=jax.ShapeDtypeStruct((N // CHUNK, 32, 4, 32), jnp.float32),
        scratch_types=[
            pltpu.VMEM((NCHUNK, CHUNK), jnp.int32),
            pltpu.VMEM((2, K, CHUNK, EMBED_DIM), jnp.float32),
            pltpu.SemaphoreType.DMA,
            pltpu.SemaphoreType.DMA,
        ],
        compiler_params=pltpu.CompilerParams(use_tc_tiling_on_sc=False),
    )(_emb_body)
    lin = f(idx, table_lin)
    out5 = _post_tiles(lin)
    out = out5.transpose(2, 4, 0, 1, 3)
    return out.reshape(BATCH, SEQ, EMBED_DIM)


# speedup vs baseline: 3.7445x; 3.7445x over previous
import functools

import jax
import jax.numpy as jnp
from jax import lax
from jax.experimental import pallas as pl
from jax.experimental.pallas import tpu as pltpu
from jax.experimental.pallas import tpu_sc as plsc

VOCAB = 1000000
EMBED_DIM = 32
BATCH = 4096
SEQ = 200

N = BATCH * SEQ
NC = 2
NS = 16
NW = NC * NS
CHUNK = 128
NCHUNK_TOT = N // CHUNK
PER_W = NCHUNK_TOT // NW
K = 4
NSUP = PER_W // K


def _emb_body(idx_hbm, table_hbm, out_hbm, idx_v, rows_v, tbuf, gsem, wsem):
    cid = lax.axis_index("c")
    sid = lax.axis_index("s")
    wid = sid * NC + cid
    cbase = wid * PER_W

    pltpu.sync_copy(idx_hbm.at[pl.ds(cbase, PER_W)], idx_v)

    iota = lax.iota(jnp.int32, 16)
    row_vecs = [iota + (16 * q) for q in range(8)]

    def transpose_chunk(p, k):
        src = rows_v.at[p, k]
        for i_tile in range(4):
            for u in range(8):
                e = 8 * i_tile + u
                col = jnp.full((16,), e, jnp.int32)
                for q in range(8):
                    vals = plsc.load_gather(src, [row_vecs[q], col])
                    tbuf[p, k, i_tile, pl.ds(u * 128 + 16 * q, 16)] = vals

    def sup_body(s, carry):
        p = lax.rem(s, 2)

        @pl.when(s >= 2)
        def _():
            for k in range(K):
                for i_tile in range(4):
                    pltpu.make_async_copy(
                        tbuf.at[p, k, i_tile], out_hbm.at[0, 0, 0], wsem
                    ).wait()

        for k in range(K):
            pltpu.async_copy(
                table_hbm.at[idx_v.at[s * K + k]], rows_v.at[p, k], gsem
            )
        for k in range(K):
            pltpu.make_async_copy(
                table_hbm.at[idx_v.at[s * K + k]], rows_v.at[p, k], gsem
            ).wait()
            transpose_chunk(p, k)
        for k in range(K):
            c = cbase + s * K + k
            s_out = lax.div(c, 32)
            j_out = lax.rem(c, 32)
            for i_tile in range(4):
                pltpu.async_copy(
                    tbuf.at[p, k, i_tile],
                    out_hbm.at[s_out, i_tile, j_out],
                    wsem,
                )
        return carry

    lax.fori_loop(0, NSUP, sup_body, 0)

    for _ in range(2 * K * 4):
        pltpu.make_async_copy(
            tbuf.at[0, 0, 0], out_hbm.at[0, 0, 0], wsem
        ).wait()


@jax.jit
def kernel(x, table):
    idx = jnp.transpose(x).reshape(NCHUNK_TOT, CHUNK).astype(jnp.int32)
    mesh = plsc.VectorSubcoreMesh(core_axis_name="c", subcore_axis_name="s")
    f = functools.partial(
        pl.kernel,
        mesh=mesh,
        out_type=jax.ShapeDtypeStruct((SEQ, 4, 32, 1024), jnp.float32),
        scratch_types=[
            pltpu.VMEM((PER_W, CHUNK), jnp.int32),
            pltpu.VMEM((2, K, CHUNK, EMBED_DIM), jnp.float32),
            pltpu.VMEM((2, K, 4, 1024), jnp.float32),
            pltpu.SemaphoreType.DMA,
            pltpu.SemaphoreType.DMA,
        ],
        compiler_params=pltpu.CompilerParams(
            use_tc_tiling_on_sc=False, needs_layout_passes=False
        ),
    )(_emb_body)
    out5 = f(idx, table)
    out = out5.reshape(SEQ, 4, 32, 8, 128).transpose(2, 4, 0, 1, 3)
    return out.reshape(BATCH, SEQ, EMBED_DIM)
